# Initial kernel scaffold; baseline (speedup 1.0000x reference)
import jax
import jax.numpy as jnp
from jax.experimental import pallas as pl


def kernel(x):
    raise NotImplementedError("write your pallas kernel here")


# speedup vs baseline: 38.3243x; 38.3243x over previous
import functools
import math

import jax
import jax.numpy as jnp
from jax import lax
from jax.experimental import pallas as pl

B = 128
N = 32768
K = math.ceil(0.25 * B)
TILE_N = 512


def _tc_body(x_ref, o_ref):
    x = x_ref[...]
    bits = lax.bitcast_convert_type(x, jnp.uint32)
    sign = bits >> 31
    u = bits ^ (jnp.uint32(0x80000000) + sign * jnp.uint32(0x7FFFFFFF))
    prefix = jnp.zeros((1, x.shape[1]), jnp.uint32)
    for b in range(31, -1, -1):
        cand = prefix | jnp.uint32(1 << b)
        cnt = jnp.sum((u >= cand).astype(jnp.int32), axis=0, keepdims=True)
        prefix = jnp.where(cnt >= K, cand, prefix)
    t = prefix
    gt = u > t
    eq = u == t
    eq_i = eq.astype(jnp.int32)
    cnt_gt = jnp.sum(gt.astype(jnp.int32), axis=0, keepdims=True)
    rem = K - cnt_gt
    incl = eq_i
    s = 1
    while s < incl.shape[0]:
        incl = incl + jnp.concatenate(
            [jnp.zeros((s, incl.shape[1]), jnp.int32), incl[:-s]], axis=0)
        s *= 2
    excl = incl - eq_i
    keep = gt | (eq & (excl < rem))
    o_ref[...] = jnp.where(keep, x, jnp.float32(0.0))


@jax.jit
def kernel(x):
    return pl.pallas_call(
        _tc_body,
        grid=(N // TILE_N,),
        in_specs=[pl.BlockSpec((B, TILE_N), lambda i: (0, i))],
        out_specs=pl.BlockSpec((B, TILE_N), lambda i: (0, i)),
        out_shape=jax.ShapeDtypeStruct((B, N), jnp.float32),
    )(x)
